# baseline (speedup 1.0000x reference)
import jax
import jax.numpy as jnp
from jax import lax
from jax.experimental import pallas as pl
from jax.experimental.pallas import tpu as pltpu
from jax.experimental.pallas import tpu_sc as plsc

N = 10000
E = 320000
F_IN = 128
HID = 128
CLS = 40
FH = F_IN // 2
CH = CLS // 2
CHP = 24

NC = 2
NS = 16
EPT = E // NS
B = 128
NB = -(-EPT // B)
EPTP = NB * B
PAD_DST = 10016
NP = 10240
RPT = NP // NS
ZCH = 128
CW = 8


def _sc_mesh():
    return plsc.VectorSubcoreMesh(
        core_axis_name="c", subcore_axis_name="s", num_cores=NC, num_subcores=NS
    )


def _sc_agg1_body(x2_hbm, edges_hbm, zrow_hbm, zcnt_hbm, ones_hbm,
                  agg_out, cnt_out,
                  src_v, dst_v, rows_v, ones_v, zrow_v, zcnt_v,
                  agg_sh, cnt_sh, gsem, ssem, osem):
    c = lax.axis_index("c")
    s = lax.axis_index("s")

    pltpu.sync_copy(zrow_hbm, zrow_v)
    pltpu.sync_copy(zcnt_hbm, zcnt_v)
    pltpu.sync_copy(ones_hbm, ones_v)
    pltpu.sync_copy(edges_hbm.at[0, s], src_v)
    pltpu.sync_copy(edges_hbm.at[1, s], dst_v)
    pad_fill = jnp.full((16,), PAD_DST, jnp.int32)
    for t in range((EPTP - EPT) // 16):
        dst_v[NB - 1, pl.ds(EPT - (NB - 1) * B + t * 16, 16)] = pad_fill

    for k in range(RPT // ZCH):
        base = s * RPT + k * ZCH
        pltpu.sync_copy(zrow_v, agg_sh.at[pl.ds(base, ZCH)])
        pltpu.sync_copy(zcnt_v, cnt_sh.at[pl.ds(base, ZCH)])
    plsc.subcore_barrier()

    for pr in range(2):
        pltpu.async_copy(x2_hbm.at[c].at[src_v.at[pr]], rows_v.at[pr], gsem)

    def body(i, carry):
        cur = lax.rem(i, 4)
        nxt = lax.rem(i + 2, 4)
        pltpu.make_async_copy(x2_hbm.at[c].at[src_v.at[i]], rows_v.at[cur], gsem).wait()

        @pl.when(i > 1)
        def _():
            pltpu.make_async_copy(rows_v.at[nxt], agg_sh.at[dst_v.at[i]],
                                  ssem).wait()

        @pl.when(i > 0)
        def _():
            pltpu.make_async_copy(ones_v, cnt_sh.at[dst_v.at[i]], osem).wait()

        @pl.when(i + 2 < NB)
        def _():
            pltpu.async_copy(x2_hbm.at[c].at[src_v.at[i + 2]], rows_v.at[nxt], gsem)

        pltpu.async_copy(rows_v.at[cur], agg_sh.at[dst_v.at[i]], ssem, add=True)
        pltpu.async_copy(ones_v, cnt_sh.at[dst_v.at[i]], osem, add=True)
        return carry

    lax.fori_loop(0, NB, body, 0)
    for tail in range(NB - 2, NB):
        pltpu.make_async_copy(rows_v.at[tail % 4], agg_sh.at[dst_v.at[tail]],
                              ssem).wait()
    pltpu.make_async_copy(ones_v, cnt_sh.at[dst_v.at[NB - 1]], osem).wait()
    plsc.subcore_barrier()

    for k in range(RPT // ZCH):
        base = s * RPT + k * ZCH
        pltpu.sync_copy(agg_sh.at[pl.ds(base, ZCH)], zrow_v)
        pltpu.sync_copy(zrow_v, agg_out.at[c, pl.ds(base, ZCH)])

    @pl.when(c == 0)
    def _():
        for k in range(RPT // ZCH):
            base = s * RPT + k * ZCH
            pltpu.sync_copy(cnt_sh.at[pl.ds(base, ZCH)], zcnt_v)
            pltpu.sync_copy(zcnt_v, cnt_out.at[pl.ds(base, ZCH)])


def _sc_agg1(x2, e4, zrow, zcnt, ones):
    return pl.kernel(
        _sc_agg1_body,
        out_type=[
            jax.ShapeDtypeStruct((NC, NP, FH), jnp.float32),
            jax.ShapeDtypeStruct((NP, CW), jnp.float32),
        ],
        mesh=_sc_mesh(),
        compiler_params=pltpu.CompilerParams(use_tc_tiling_on_sc=False),
        scratch_types=[
            pltpu.VMEM((NB, B), jnp.int32),
            pltpu.VMEM((NB, B), jnp.int32),
            pltpu.VMEM((4, B, FH), jnp.float32),
            pltpu.VMEM((B, CW), jnp.float32),
            pltpu.VMEM((ZCH, FH), jnp.float32),
            pltpu.VMEM((ZCH, CW), jnp.float32),
            pltpu.VMEM_SHARED((NP, FH), jnp.float32),
            pltpu.VMEM_SHARED((NP, CW), jnp.float32),
            pltpu.SemaphoreType.DMA,
            pltpu.SemaphoreType.DMA,
            pltpu.SemaphoreType.DMA,
        ],
    )(x2, e4, zrow, zcnt, ones)


def _sc_agg2_body(p_hbm, edges_hbm, zrow_hbm,
                  agg_out,
                  src_v, dst_v, rows_v, zrow_v,
                  agg_sh, gsem, ssem):
    c = lax.axis_index("c")
    s = lax.axis_index("s")

    pltpu.sync_copy(zrow_hbm, zrow_v)
    pltpu.sync_copy(edges_hbm.at[0, s], src_v)
    pltpu.sync_copy(edges_hbm.at[1, s], dst_v)
    pad_fill = jnp.full((16,), PAD_DST, jnp.int32)
    for t in range((EPTP - EPT) // 16):
        dst_v[NB - 1, pl.ds(EPT - (NB - 1) * B + t * 16, 16)] = pad_fill

    for k in range(RPT // ZCH):
        base = s * RPT + k * ZCH
        pltpu.sync_copy(zrow_v, agg_sh.at[pl.ds(base, ZCH)])
    plsc.subcore_barrier()

    for pr in range(4):
        pltpu.async_copy(p_hbm.at[c].at[src_v.at[pr]], rows_v.at[pr], gsem)

    def body(i, carry):
        cur = lax.rem(i, 8)
        nxt = lax.rem(i + 4, 8)
        pltpu.make_async_copy(p_hbm.at[c].at[src_v.at[i]], rows_v.at[cur], gsem).wait()

        @pl.when(i > 3)
        def _():
            pltpu.make_async_copy(rows_v.at[nxt], agg_sh.at[dst_v.at[i]],
                                  ssem).wait()

        @pl.when(i + 4 < NB)
        def _():
            pltpu.async_copy(p_hbm.at[c].at[src_v.at[i + 4]], rows_v.at[nxt], gsem)

        pltpu.async_copy(rows_v.at[cur], agg_sh.at[dst_v.at[i]], ssem, add=True)
        return carry

    lax.fori_loop(0, NB, body, 0)
    for tail in range(NB - 4, NB):
        pltpu.make_async_copy(rows_v.at[tail % 8], agg_sh.at[dst_v.at[tail]],
                              ssem).wait()
    plsc.subcore_barrier()

    for k in range(RPT // ZCH):
        base = s * RPT + k * ZCH
        pltpu.sync_copy(agg_sh.at[pl.ds(base, ZCH)], zrow_v)
        pltpu.sync_copy(zrow_v, agg_out.at[c, pl.ds(base, ZCH)])


def _sc_agg2(p2, e4, zrow):
    return pl.kernel(
        _sc_agg2_body,
        out_type=jax.ShapeDtypeStruct((NC, NP, CHP), jnp.float32),
        mesh=_sc_mesh(),
        compiler_params=pltpu.CompilerParams(use_tc_tiling_on_sc=False),
        scratch_types=[
            pltpu.VMEM((NB, B), jnp.int32),
            pltpu.VMEM((NB, B), jnp.int32),
            pltpu.VMEM((8, B, CHP), jnp.float32),
            pltpu.VMEM((ZCH, CHP), jnp.float32),
            pltpu.VMEM_SHARED((NP, CHP), jnp.float32),
            pltpu.SemaphoreType.DMA,
            pltpu.SemaphoreType.DMA,
        ],
    )(p2, e4, zrow)


RB = 1000


def _dot_t(a, w):
    return lax.dot_general(a, w, (((1,), (1,)), ((), ())),
                           preferred_element_type=jnp.float32)


def _tc_layer1_body(aL, aR, cn, x, w1l, b1l, w1r, w2l, b2l, w2r, p2_out, q_out):
    agg = jnp.concatenate([aL[0], aR[0]], axis=1)
    cnt = jnp.maximum(cn[...], 1.0)
    mean = agg / cnt
    h = _dot_t(mean, w1l[...]) + b1l[...] + _dot_t(x[...], w1r[...])
    h = jnp.maximum(h, 0.0)
    p = _dot_t(h, w2l[...])
    zpad = jnp.zeros((p.shape[0], CHP - CH), jnp.float32)
    p2_out[0] = jnp.concatenate([p[:, :CH], zpad], axis=1)
    p2_out[1] = jnp.concatenate([p[:, CH:], zpad], axis=1)
    q_out[...] = _dot_t(h, w2r[...]) + b2l[...]


def _tc_layer1(agg_pp, cn, x, W1l, b1l, W1r, W2l, b2l, W2r):
    blk = lambda r, c: pl.BlockSpec((r, c), lambda i: (i, 0))
    full = lambda r, c: pl.BlockSpec((r, c), lambda i: (0, 0))
    return pl.pallas_call(
        _tc_layer1_body,
        grid=(N // RB,),
        in_specs=[
            pl.BlockSpec((1, RB, FH), lambda i: (0, i, 0)),
            pl.BlockSpec((1, RB, FH), lambda i: (1, i, 0)),
            blk(RB, 1), blk(RB, F_IN),
            full(HID, F_IN), full(1, HID), full(HID, F_IN),
            full(CLS, HID), full(1, CLS), full(CLS, HID),
        ],
        out_specs=[
            pl.BlockSpec((NC, RB, CHP), lambda i: (0, i, 0)),
            blk(RB, CLS),
        ],
        out_shape=[
            jax.ShapeDtypeStruct((NC, N, CHP), jnp.float32),
            jax.ShapeDtypeStruct((N, CLS), jnp.float32),
        ],
    )(agg_pp, agg_pp, cn, x, W1l, b1l, W1r, W2l, b2l, W2r)


def _tc_layer2_body(g0, g1, cn, q, out):
    agg = jnp.concatenate([g0[0, :, :CH], g1[0, :, :CH]], axis=1)
    cnt = jnp.maximum(cn[...], 1.0)
    z = agg / cnt + q[...]
    m = jnp.max(z, axis=1, keepdims=True)
    zs = z - m
    out[...] = zs - jnp.log(jnp.sum(jnp.exp(zs), axis=1, keepdims=True))


def _tc_layer2(agg2_pp, cn, q):
    blk = lambda r, c: pl.BlockSpec((r, c), lambda i: (i, 0))
    return pl.pallas_call(
        _tc_layer2_body,
        grid=(N // RB,),
        in_specs=[
            pl.BlockSpec((1, RB, CHP), lambda i: (0, i, 0)),
            pl.BlockSpec((1, RB, CHP), lambda i: (1, i, 0)),
            blk(RB, 1), blk(RB, CLS),
        ],
        out_specs=blk(RB, CLS),
        out_shape=jax.ShapeDtypeStruct((N, CLS), jnp.float32),
    )(agg2_pp, agg2_pp, cn, q)


def kernel(x, edge_index, W1l, b1l, W1r, W2l, b2l, W2r):
    e4 = jnp.pad(edge_index.reshape(2, NS, EPT),
                 ((0, 0), (0, 0), (0, EPTP - EPT))).reshape(2, NS, NB, B)
    x2 = jnp.stack([x[:, :FH], x[:, FH:]])

    zrow = jnp.zeros((ZCH, FH), jnp.float32)
    zcnt = jnp.zeros((ZCH, CW), jnp.float32)
    zrow2 = jnp.zeros((ZCH, CHP), jnp.float32)
    ones = jnp.ones((B, CW), jnp.float32)

    agg_pp, cnt_pp = _sc_agg1(x2, e4, zrow, zcnt, ones)
    cn = cnt_pp[:N, :1]
    p2, q = _tc_layer1(agg_pp, cn, x,
                       W1l, b1l.reshape(1, HID), W1r,
                       W2l, b2l.reshape(1, CLS), W2r)

    agg2_pp = _sc_agg2(p2, e4, zrow2)
    return _tc_layer2(agg2_pp, cn, q)


# speedup vs baseline: 1.0009x; 1.0009x over previous
import jax
import jax.numpy as jnp
from jax import lax
from jax.experimental import pallas as pl
from jax.experimental.pallas import tpu as pltpu
from jax.experimental.pallas import tpu_sc as plsc

N = 10000
E = 320000
F_IN = 128
HID = 128
CLS = 40
FH = F_IN // 2
CH = CLS // 2
CHP = 24

NC = 2
NS = 16
EPT = E // NS
B = 128
NB = -(-EPT // B)
EPTP = NB * B
PAD_DST = 10016
NP = 10240
RPT = NP // NS
ZCH = 128
CW = 8


def _sc_mesh():
    return plsc.VectorSubcoreMesh(
        core_axis_name="c", subcore_axis_name="s", num_cores=NC, num_subcores=NS
    )


def _sc_agg1_body(x2_hbm, edges_hbm, zrow_hbm, zcnt_hbm, ones_hbm,
                  agg_out, cnt_out,
                  src_v, dst_v, rows_v, ones_v, zrow_v, zcnt_v,
                  agg_sh, cnt_sh, gsem, ssem, osem):
    c = lax.axis_index("c")
    s = lax.axis_index("s")

    pltpu.sync_copy(zrow_hbm, zrow_v)
    pltpu.sync_copy(zcnt_hbm, zcnt_v)
    pltpu.sync_copy(ones_hbm, ones_v)
    pltpu.sync_copy(edges_hbm.at[0, s], src_v)
    pltpu.sync_copy(edges_hbm.at[1, s], dst_v)
    pad_fill = jnp.full((16,), PAD_DST, jnp.int32)
    for t in range((EPTP - EPT) // 16):
        dst_v[NB - 1, pl.ds(EPT - (NB - 1) * B + t * 16, 16)] = pad_fill

    for k in range(RPT // ZCH):
        base = s * RPT + k * ZCH
        pltpu.sync_copy(zrow_v, agg_sh.at[pl.ds(base, ZCH)])
        pltpu.sync_copy(zcnt_v, cnt_sh.at[pl.ds(base, ZCH)])
    plsc.subcore_barrier()

    for pr in range(2):
        pltpu.async_copy(x2_hbm.at[c].at[src_v.at[pr]], rows_v.at[pr], gsem)

    def body(i, carry):
        cur = lax.rem(i, 4)
        nxt = lax.rem(i + 2, 4)
        pltpu.make_async_copy(x2_hbm.at[c].at[src_v.at[i]], rows_v.at[cur], gsem).wait()

        @pl.when(i > 1)
        def _():
            pltpu.make_async_copy(rows_v.at[nxt], agg_sh.at[dst_v.at[i]],
                                  ssem).wait()

        @pl.when(i > 1)
        def _():
            pltpu.make_async_copy(ones_v, cnt_sh.at[dst_v.at[i]], osem).wait()

        @pl.when(i + 2 < NB)
        def _():
            pltpu.async_copy(x2_hbm.at[c].at[src_v.at[i + 2]], rows_v.at[nxt], gsem)

        pltpu.async_copy(rows_v.at[cur], agg_sh.at[dst_v.at[i]], ssem, add=True)
        pltpu.async_copy(ones_v, cnt_sh.at[dst_v.at[i]], osem, add=True)
        return carry

    lax.fori_loop(0, NB, body, 0)
    for tail in range(NB - 2, NB):
        pltpu.make_async_copy(rows_v.at[tail % 4], agg_sh.at[dst_v.at[tail]],
                              ssem).wait()
    for tail in range(NB - 2, NB):
        pltpu.make_async_copy(ones_v, cnt_sh.at[dst_v.at[tail]], osem).wait()
    plsc.subcore_barrier()

    for k in range(RPT // ZCH):
        base = s * RPT + k * ZCH
        pltpu.sync_copy(agg_sh.at[pl.ds(base, ZCH)], zrow_v)
        pltpu.sync_copy(zrow_v, agg_out.at[c, pl.ds(base, ZCH)])

    @pl.when(c == 0)
    def _():
        for k in range(RPT // ZCH):
            base = s * RPT + k * ZCH
            pltpu.sync_copy(cnt_sh.at[pl.ds(base, ZCH)], zcnt_v)
            pltpu.sync_copy(zcnt_v, cnt_out.at[pl.ds(base, ZCH)])


def _sc_agg1(x2, e4, zrow, zcnt, ones):
    return pl.kernel(
        _sc_agg1_body,
        out_type=[
            jax.ShapeDtypeStruct((NC, NP, FH), jnp.float32),
            jax.ShapeDtypeStruct((NP, CW), jnp.float32),
        ],
        mesh=_sc_mesh(),
        compiler_params=pltpu.CompilerParams(use_tc_tiling_on_sc=False),
        scratch_types=[
            pltpu.VMEM((NB, B), jnp.int32),
            pltpu.VMEM((NB, B), jnp.int32),
            pltpu.VMEM((4, B, FH), jnp.float32),
            pltpu.VMEM((B, CW), jnp.float32),
            pltpu.VMEM((ZCH, FH), jnp.float32),
            pltpu.VMEM((ZCH, CW), jnp.float32),
            pltpu.VMEM_SHARED((NP, FH), jnp.float32),
            pltpu.VMEM_SHARED((NP, CW), jnp.float32),
            pltpu.SemaphoreType.DMA,
            pltpu.SemaphoreType.DMA,
            pltpu.SemaphoreType.DMA,
        ],
    )(x2, e4, zrow, zcnt, ones)


def _sc_agg2_body(p_hbm, edges_hbm, zrow_hbm,
                  agg_out,
                  src_v, dst_v, rows_v, zrow_v,
                  agg_sh, gsem, ssem):
    c = lax.axis_index("c")
    s = lax.axis_index("s")

    pltpu.sync_copy(zrow_hbm, zrow_v)
    pltpu.sync_copy(edges_hbm.at[0, s], src_v)
    pltpu.sync_copy(edges_hbm.at[1, s], dst_v)
    pad_fill = jnp.full((16,), PAD_DST, jnp.int32)
    for t in range((EPTP - EPT) // 16):
        dst_v[NB - 1, pl.ds(EPT - (NB - 1) * B + t * 16, 16)] = pad_fill

    for k in range(RPT // ZCH):
        base = s * RPT + k * ZCH
        pltpu.sync_copy(zrow_v, agg_sh.at[pl.ds(base, ZCH)])
    plsc.subcore_barrier()

    for pr in range(4):
        pltpu.async_copy(p_hbm.at[c].at[src_v.at[pr]], rows_v.at[pr], gsem)

    def body(i, carry):
        cur = lax.rem(i, 8)
        nxt = lax.rem(i + 4, 8)
        pltpu.make_async_copy(p_hbm.at[c].at[src_v.at[i]], rows_v.at[cur], gsem).wait()

        @pl.when(i > 3)
        def _():
            pltpu.make_async_copy(rows_v.at[nxt], agg_sh.at[dst_v.at[i]],
                                  ssem).wait()

        @pl.when(i + 4 < NB)
        def _():
            pltpu.async_copy(p_hbm.at[c].at[src_v.at[i + 4]], rows_v.at[nxt], gsem)

        pltpu.async_copy(rows_v.at[cur], agg_sh.at[dst_v.at[i]], ssem, add=True)
        return carry

    lax.fori_loop(0, NB, body, 0)
    for tail in range(NB - 4, NB):
        pltpu.make_async_copy(rows_v.at[tail % 8], agg_sh.at[dst_v.at[tail]],
                              ssem).wait()
    plsc.subcore_barrier()

    for k in range(RPT // ZCH):
        base = s * RPT + k * ZCH
        pltpu.sync_copy(agg_sh.at[pl.ds(base, ZCH)], zrow_v)
        pltpu.sync_copy(zrow_v, agg_out.at[c, pl.ds(base, ZCH)])


def _sc_agg2(p2, e4, zrow):
    return pl.kernel(
        _sc_agg2_body,
        out_type=jax.ShapeDtypeStruct((NC, NP, CHP), jnp.float32),
        mesh=_sc_mesh(),
        compiler_params=pltpu.CompilerParams(use_tc_tiling_on_sc=False),
        scratch_types=[
            pltpu.VMEM((NB, B), jnp.int32),
            pltpu.VMEM((NB, B), jnp.int32),
            pltpu.VMEM((8, B, CHP), jnp.float32),
            pltpu.VMEM((ZCH, CHP), jnp.float32),
            pltpu.VMEM_SHARED((NP, CHP), jnp.float32),
            pltpu.SemaphoreType.DMA,
            pltpu.SemaphoreType.DMA,
        ],
    )(p2, e4, zrow)


RB = 1000


def _dot_t(a, w):
    return lax.dot_general(a, w, (((1,), (1,)), ((), ())),
                           preferred_element_type=jnp.float32)


def _tc_layer1_body(aL, aR, cn, x, w1l, b1l, w1r, w2l, b2l, w2r, p2_out, q_out):
    agg = jnp.concatenate([aL[0], aR[0]], axis=1)
    cnt = jnp.maximum(cn[...], 1.0)
    mean = agg / cnt
    h = _dot_t(mean, w1l[...]) + b1l[...] + _dot_t(x[...], w1r[...])
    h = jnp.maximum(h, 0.0)
    p = _dot_t(h, w2l[...])
    zpad = jnp.zeros((p.shape[0], CHP - CH), jnp.float32)
    p2_out[0] = jnp.concatenate([p[:, :CH], zpad], axis=1)
    p2_out[1] = jnp.concatenate([p[:, CH:], zpad], axis=1)
    q_out[...] = _dot_t(h, w2r[...]) + b2l[...]


def _tc_layer1(agg_pp, cn, x, W1l, b1l, W1r, W2l, b2l, W2r):
    blk = lambda r, c: pl.BlockSpec((r, c), lambda i: (i, 0))
    full = lambda r, c: pl.BlockSpec((r, c), lambda i: (0, 0))
    return pl.pallas_call(
        _tc_layer1_body,
        grid=(N // RB,),
        in_specs=[
            pl.BlockSpec((1, RB, FH), lambda i: (0, i, 0)),
            pl.BlockSpec((1, RB, FH), lambda i: (1, i, 0)),
            blk(RB, 1), blk(RB, F_IN),
            full(HID, F_IN), full(1, HID), full(HID, F_IN),
            full(CLS, HID), full(1, CLS), full(CLS, HID),
        ],
        out_specs=[
            pl.BlockSpec((NC, RB, CHP), lambda i: (0, i, 0)),
            blk(RB, CLS),
        ],
        out_shape=[
            jax.ShapeDtypeStruct((NC, N, CHP), jnp.float32),
            jax.ShapeDtypeStruct((N, CLS), jnp.float32),
        ],
    )(agg_pp, agg_pp, cn, x, W1l, b1l, W1r, W2l, b2l, W2r)


def _tc_layer2_body(g0, g1, cn, q, out):
    agg = jnp.concatenate([g0[0, :, :CH], g1[0, :, :CH]], axis=1)
    cnt = jnp.maximum(cn[...], 1.0)
    z = agg / cnt + q[...]
    m = jnp.max(z, axis=1, keepdims=True)
    zs = z - m
    out[...] = zs - jnp.log(jnp.sum(jnp.exp(zs), axis=1, keepdims=True))


def _tc_layer2(agg2_pp, cn, q):
    blk = lambda r, c: pl.BlockSpec((r, c), lambda i: (i, 0))
    return pl.pallas_call(
        _tc_layer2_body,
        grid=(N // RB,),
        in_specs=[
            pl.BlockSpec((1, RB, CHP), lambda i: (0, i, 0)),
            pl.BlockSpec((1, RB, CHP), lambda i: (1, i, 0)),
            blk(RB, 1), blk(RB, CLS),
        ],
        out_specs=blk(RB, CLS),
        out_shape=jax.ShapeDtypeStruct((N, CLS), jnp.float32),
    )(agg2_pp, agg2_pp, cn, q)


def kernel(x, edge_index, W1l, b1l, W1r, W2l, b2l, W2r):
    e4 = jnp.pad(edge_index.reshape(2, NS, EPT),
                 ((0, 0), (0, 0), (0, EPTP - EPT))).reshape(2, NS, NB, B)
    x2 = jnp.stack([x[:, :FH], x[:, FH:]])

    zrow = jnp.zeros((ZCH, FH), jnp.float32)
    zcnt = jnp.zeros((ZCH, CW), jnp.float32)
    zrow2 = jnp.zeros((ZCH, CHP), jnp.float32)
    ones = jnp.ones((B, CW), jnp.float32)

    agg_pp, cnt_pp = _sc_agg1(x2, e4, zrow, zcnt, ones)
    cn = cnt_pp[:N, :1]
    p2, q = _tc_layer1(agg_pp, cn, x,
                       W1l, b1l.reshape(1, HID), W1r,
                       W2l, b2l.reshape(1, CLS), W2r)

    agg2_pp = _sc_agg2(p2, e4, zrow2)
    return _tc_layer2(agg2_pp, cn, q)


# speedup vs baseline: 1.0036x; 1.0026x over previous
import jax
import jax.numpy as jnp
from jax import lax
from jax.experimental import pallas as pl
from jax.experimental.pallas import tpu as pltpu
from jax.experimental.pallas import tpu_sc as plsc

N = 10000
E = 320000
F_IN = 128
HID = 128
CLS = 40
FH = F_IN // 2
CH = CLS // 2
CHP = 24

NC = 2
NS = 16
EPT = E // NS
B = 128
NB = -(-EPT // B)
EPTP = NB * B
PAD_DST = 10016
NP = 10240
RPT = NP // NS
ZCH = 128
CW = 8


def _sc_mesh():
    return plsc.VectorSubcoreMesh(
        core_axis_name="c", subcore_axis_name="s", num_cores=NC, num_subcores=NS
    )


def _sc_agg1_body(x2_hbm, edges_hbm, zrow_hbm, zcnt_hbm, ones_hbm,
                  agg_out, cnt_out,
                  src_v, dst_v, rows_v, ones_v, zrow_v, zcnt_v,
                  agg_sh, cnt_sh, gsem, ssem, osem):
    c = lax.axis_index("c")
    s = lax.axis_index("s")

    pltpu.sync_copy(zrow_hbm, zrow_v)
    pltpu.sync_copy(zcnt_hbm, zcnt_v)
    pltpu.sync_copy(ones_hbm, ones_v)
    pltpu.sync_copy(edges_hbm.at[0, s], src_v)
    pltpu.sync_copy(edges_hbm.at[1, s], dst_v)
    pad_fill = jnp.full((16,), PAD_DST, jnp.int32)
    for t in range((EPTP - EPT) // 16):
        dst_v[NB - 1, pl.ds(EPT - (NB - 1) * B + t * 16, 16)] = pad_fill

    for k in range(RPT // ZCH):
        base = s * RPT + k * ZCH
        pltpu.sync_copy(zrow_v, agg_sh.at[pl.ds(base, ZCH)])
        pltpu.sync_copy(zcnt_v, cnt_sh.at[pl.ds(base, ZCH)])
    plsc.subcore_barrier()

    for pr in range(2):
        pltpu.async_copy(x2_hbm.at[c].at[src_v.at[pr]], rows_v.at[pr], gsem)

    def body(i, carry):
        cur = lax.rem(i, 4)
        nxt = lax.rem(i + 2, 4)
        pltpu.make_async_copy(x2_hbm.at[c].at[src_v.at[i]], rows_v.at[cur], gsem).wait()

        @pl.when(i > 1)
        def _():
            pltpu.make_async_copy(rows_v.at[nxt], agg_sh.at[dst_v.at[i]],
                                  ssem).wait()

        @pl.when(i > 1)
        def _():
            pltpu.make_async_copy(ones_v, cnt_sh.at[dst_v.at[i]], osem).wait()

        @pl.when(i + 2 < NB)
        def _():
            pltpu.async_copy(x2_hbm.at[c].at[src_v.at[i + 2]], rows_v.at[nxt], gsem)

        pltpu.async_copy(rows_v.at[cur], agg_sh.at[dst_v.at[i]], ssem, add=True)
        pltpu.async_copy(ones_v, cnt_sh.at[dst_v.at[i]], osem, add=True)
        return carry

    lax.fori_loop(0, NB, body, 0)
    for tail in range(NB - 2, NB):
        pltpu.make_async_copy(rows_v.at[tail % 4], agg_sh.at[dst_v.at[tail]],
                              ssem).wait()
    for tail in range(NB - 2, NB):
        pltpu.make_async_copy(ones_v, cnt_sh.at[dst_v.at[tail]], osem).wait()
    plsc.subcore_barrier()

    for k in range(RPT // ZCH):
        base = s * RPT + k * ZCH
        pltpu.sync_copy(agg_sh.at[pl.ds(base, ZCH)], zrow_v)
        pltpu.sync_copy(zrow_v, agg_out.at[c, pl.ds(base, ZCH)])

    @pl.when(c == 0)
    def _():
        for k in range(RPT // ZCH):
            base = s * RPT + k * ZCH
            pltpu.sync_copy(cnt_sh.at[pl.ds(base, ZCH)], zcnt_v)
            pltpu.sync_copy(zcnt_v, cnt_out.at[pl.ds(base, ZCH)])


def _sc_agg1(x2, e4, zrow, zcnt, ones):
    return pl.kernel(
        _sc_agg1_body,
        out_type=[
            jax.ShapeDtypeStruct((NC, NP, FH), jnp.float32),
            jax.ShapeDtypeStruct((NP, CW), jnp.float32),
        ],
        mesh=_sc_mesh(),
        compiler_params=pltpu.CompilerParams(use_tc_tiling_on_sc=False),
        scratch_types=[
            pltpu.VMEM((NB, B), jnp.int32),
            pltpu.VMEM((NB, B), jnp.int32),
            pltpu.VMEM((4, B, FH), jnp.float32),
            pltpu.VMEM((B, CW), jnp.float32),
            pltpu.VMEM((ZCH, FH), jnp.float32),
            pltpu.VMEM((ZCH, CW), jnp.float32),
            pltpu.VMEM_SHARED((NP, FH), jnp.float32),
            pltpu.VMEM_SHARED((NP, CW), jnp.float32),
            pltpu.SemaphoreType.DMA,
            pltpu.SemaphoreType.DMA,
            pltpu.SemaphoreType.DMA,
        ],
    )(x2, e4, zrow, zcnt, ones)


def _sc_agg2_body(p_hbm, edges_hbm, zrow_hbm,
                  agg_out,
                  src_v, dst_v, rows_v, zrow_v,
                  agg_sh, gsem, ssem):
    c = lax.axis_index("c")
    s = lax.axis_index("s")

    pltpu.sync_copy(zrow_hbm, zrow_v)
    pltpu.sync_copy(edges_hbm.at[0, s], src_v)
    pltpu.sync_copy(edges_hbm.at[1, s], dst_v)
    pad_fill = jnp.full((16,), PAD_DST, jnp.int32)
    for t in range((EPTP - EPT) // 16):
        dst_v[NB - 1, pl.ds(EPT - (NB - 1) * B + t * 16, 16)] = pad_fill

    for k in range(RPT // ZCH):
        base = s * RPT + k * ZCH
        pltpu.sync_copy(zrow_v, agg_sh.at[pl.ds(base, ZCH)])
    plsc.subcore_barrier()

    for pr in range(4):
        pltpu.async_copy(p_hbm.at[c].at[src_v.at[pr]], rows_v.at[pr], gsem)

    def body(i, carry):
        cur = lax.rem(i, 8)
        nxt = lax.rem(i + 4, 8)
        pltpu.make_async_copy(p_hbm.at[c].at[src_v.at[i]], rows_v.at[cur], gsem).wait()

        @pl.when(i > 3)
        def _():
            pltpu.make_async_copy(rows_v.at[nxt], agg_sh.at[dst_v.at[i]],
                                  ssem).wait()

        @pl.when(i + 4 < NB)
        def _():
            pltpu.async_copy(p_hbm.at[c].at[src_v.at[i + 4]], rows_v.at[nxt], gsem)

        pltpu.async_copy(rows_v.at[cur], agg_sh.at[dst_v.at[i]], ssem, add=True)
        return carry

    lax.fori_loop(0, NB, body, 0)
    for tail in range(NB - 4, NB):
        pltpu.make_async_copy(rows_v.at[tail % 8], agg_sh.at[dst_v.at[tail]],
                              ssem).wait()
    plsc.subcore_barrier()

    for k in range(RPT // ZCH):
        base = s * RPT + k * ZCH
        pltpu.sync_copy(agg_sh.at[pl.ds(base, ZCH)], zrow_v)
        pltpu.sync_copy(zrow_v, agg_out.at[c, pl.ds(base, ZCH)])


def _sc_agg2(p2, e4, zrow):
    return pl.kernel(
        _sc_agg2_body,
        out_type=jax.ShapeDtypeStruct((NC, NP, CHP), jnp.float32),
        mesh=_sc_mesh(),
        compiler_params=pltpu.CompilerParams(use_tc_tiling_on_sc=False),
        scratch_types=[
            pltpu.VMEM((NB, B), jnp.int32),
            pltpu.VMEM((NB, B), jnp.int32),
            pltpu.VMEM((8, B, CHP), jnp.float32),
            pltpu.VMEM((ZCH, CHP), jnp.float32),
            pltpu.VMEM_SHARED((NP, CHP), jnp.float32),
            pltpu.SemaphoreType.DMA,
            pltpu.SemaphoreType.DMA,
        ],
    )(p2, e4, zrow)


RB = 1000


def _dot_t(a, w):
    return lax.dot_general(a, w, (((1,), (1,)), ((), ())),
                           preferred_element_type=jnp.float32)


def _tc_layer1_body(aL, aR, cn, x, w1, b1l, w2, b2l, p2_out, q_out):
    cnt = jnp.maximum(cn[...], 1.0)
    mean = jnp.concatenate([aL[0], aR[0]], axis=1) / cnt
    mx = jnp.concatenate([mean, x[...]], axis=1)
    h = jnp.maximum(_dot_t(mx, w1[...]) + b1l[...], 0.0)
    pq = _dot_t(h, w2[...])
    zpad = jnp.zeros((pq.shape[0], CHP - CH), jnp.float32)
    p2_out[0] = jnp.concatenate([pq[:, :CH], zpad], axis=1)
    p2_out[1] = jnp.concatenate([pq[:, CH:CLS], zpad], axis=1)
    q_out[...] = pq[:, CLS:] + b2l[...]


def _tc_layer1(agg_pp, cn, x, W1l, b1l, W1r, W2l, b2l, W2r):
    W1 = jnp.concatenate([W1l, W1r], axis=1)
    W2 = jnp.concatenate([W2l, W2r], axis=0)
    blk = lambda r, c: pl.BlockSpec((r, c), lambda i: (i, 0))
    full = lambda r, c: pl.BlockSpec((r, c), lambda i: (0, 0))
    return pl.pallas_call(
        _tc_layer1_body,
        grid=(N // RB,),
        in_specs=[
            pl.BlockSpec((1, RB, FH), lambda i: (0, i, 0)),
            pl.BlockSpec((1, RB, FH), lambda i: (1, i, 0)),
            blk(RB, 1), blk(RB, F_IN),
            full(HID, 2 * F_IN), full(1, HID),
            full(2 * CLS, HID), full(1, CLS),
        ],
        out_specs=[
            pl.BlockSpec((NC, RB, CHP), lambda i: (0, i, 0)),
            blk(RB, CLS),
        ],
        out_shape=[
            jax.ShapeDtypeStruct((NC, N, CHP), jnp.float32),
            jax.ShapeDtypeStruct((N, CLS), jnp.float32),
        ],
    )(agg_pp, agg_pp, cn, x, W1, b1l, W2, b2l)


def _tc_layer2_body(g0, g1, cn, q, out):
    agg = jnp.concatenate([g0[0, :, :CH], g1[0, :, :CH]], axis=1)
    cnt = jnp.maximum(cn[...], 1.0)
    z = agg / cnt + q[...]
    m = jnp.max(z, axis=1, keepdims=True)
    zs = z - m
    out[...] = zs - jnp.log(jnp.sum(jnp.exp(zs), axis=1, keepdims=True))


def _tc_layer2(agg2_pp, cn, q):
    blk = lambda r, c: pl.BlockSpec((r, c), lambda i: (i, 0))
    return pl.pallas_call(
        _tc_layer2_body,
        grid=(N // RB,),
        in_specs=[
            pl.BlockSpec((1, RB, CHP), lambda i: (0, i, 0)),
            pl.BlockSpec((1, RB, CHP), lambda i: (1, i, 0)),
            blk(RB, 1), blk(RB, CLS),
        ],
        out_specs=blk(RB, CLS),
        out_shape=jax.ShapeDtypeStruct((N, CLS), jnp.float32),
    )(agg2_pp, agg2_pp, cn, q)


def kernel(x, edge_index, W1l, b1l, W1r, W2l, b2l, W2r):
    e4 = jnp.pad(edge_index.reshape(2, NS, EPT),
                 ((0, 0), (0, 0), (0, EPTP - EPT))).reshape(2, NS, NB, B)
    x2 = jnp.stack([x[:, :FH], x[:, FH:]])

    zrow = jnp.zeros((ZCH, FH), jnp.float32)
    zcnt = jnp.zeros((ZCH, CW), jnp.float32)
    zrow2 = jnp.zeros((ZCH, CHP), jnp.float32)
    ones = jnp.ones((B, CW), jnp.float32)

    agg_pp, cnt_pp = _sc_agg1(x2, e4, zrow, zcnt, ones)
    cn = cnt_pp[:N, :1]
    p2, q = _tc_layer1(agg_pp, cn, x,
                       W1l, b1l.reshape(1, HID), W1r,
                       W2l, b2l.reshape(1, CLS), W2r)

    agg2_pp = _sc_agg2(p2, e4, zrow2)
    return _tc_layer2(agg2_pp, cn, q)


# speedup vs baseline: 1.0129x; 1.0093x over previous
import jax
import jax.numpy as jnp
from jax import lax
from jax.experimental import pallas as pl
from jax.experimental.pallas import tpu as pltpu
from jax.experimental.pallas import tpu_sc as plsc

N = 10000
E = 320000
F_IN = 128
HID = 128
CLS = 40
FH = F_IN // 2
CH = CLS // 2
CHP = 24

NC = 2
NS = 16
EPT = E // NS
B = 128
NB = -(-EPT // B)
EPTP = NB * B
PAD_DST = 10016
NP = 10240
RPT = NP // NS
ZCH = 128
CW = 8


def _sc_mesh():
    return plsc.VectorSubcoreMesh(
        core_axis_name="c", subcore_axis_name="s", num_cores=NC, num_subcores=NS
    )


def _sc_agg1_body(x2_hbm, edges_hbm, zrow_hbm, zcnt_hbm, ones_hbm,
                  agg_out, cnt_out,
                  src_v, dst_v, rows_v, ones_v, zrow_v, zcnt_v,
                  agg_sh, cnt_sh, gsem, ssem, osem):
    c = lax.axis_index("c")
    s = lax.axis_index("s")

    pltpu.sync_copy(zrow_hbm, zrow_v)
    pltpu.sync_copy(zcnt_hbm, zcnt_v)
    pltpu.sync_copy(ones_hbm, ones_v)
    pltpu.sync_copy(edges_hbm.at[0, s], src_v)
    pltpu.sync_copy(edges_hbm.at[1, s], dst_v)
    pad_fill = jnp.full((16,), PAD_DST, jnp.int32)
    for t in range((EPTP - EPT) // 16):
        dst_v[NB - 1, pl.ds(EPT - (NB - 1) * B + t * 16, 16)] = pad_fill

    for k in range(RPT // ZCH):
        base = s * RPT + k * ZCH
        pltpu.async_copy(zrow_v, agg_sh.at[pl.ds(base, ZCH)], gsem)
        pltpu.async_copy(zcnt_v, cnt_sh.at[pl.ds(base, ZCH)], ssem)
    for k in range(RPT // ZCH):
        base = s * RPT + k * ZCH
        pltpu.make_async_copy(zrow_v, agg_sh.at[pl.ds(base, ZCH)], gsem).wait()
        pltpu.make_async_copy(zcnt_v, cnt_sh.at[pl.ds(base, ZCH)], ssem).wait()
    plsc.subcore_barrier()

    for pr in range(2):
        pltpu.async_copy(x2_hbm.at[c].at[src_v.at[pr]], rows_v.at[pr], gsem)

    def body(i, carry):
        cur = lax.rem(i, 4)
        nxt = lax.rem(i + 2, 4)
        pltpu.make_async_copy(x2_hbm.at[c].at[src_v.at[i]], rows_v.at[cur], gsem).wait()

        @pl.when(i > 1)
        def _():
            pltpu.make_async_copy(rows_v.at[nxt], agg_sh.at[dst_v.at[i]],
                                  ssem).wait()

        @pl.when(i > 1)
        def _():
            pltpu.make_async_copy(ones_v, cnt_sh.at[dst_v.at[i]], osem).wait()

        @pl.when(i + 2 < NB)
        def _():
            pltpu.async_copy(x2_hbm.at[c].at[src_v.at[i + 2]], rows_v.at[nxt], gsem)

        pltpu.async_copy(rows_v.at[cur], agg_sh.at[dst_v.at[i]], ssem, add=True)
        pltpu.async_copy(ones_v, cnt_sh.at[dst_v.at[i]], osem, add=True)
        return carry

    lax.fori_loop(0, NB, body, 0)
    for tail in range(NB - 2, NB):
        pltpu.make_async_copy(rows_v.at[tail % 4], agg_sh.at[dst_v.at[tail]],
                              ssem).wait()
    for tail in range(NB - 2, NB):
        pltpu.make_async_copy(ones_v, cnt_sh.at[dst_v.at[tail]], osem).wait()
    plsc.subcore_barrier()

    ck = [pl.ds(s * RPT + k * ZCH, ZCH) for k in range(RPT // ZCH)]
    for k in range(4):
        pltpu.async_copy(agg_sh.at[ck[k]], rows_v.at[k], gsem)
    for k in range(4):
        pltpu.make_async_copy(agg_sh.at[ck[k]], rows_v.at[k], gsem).wait()
        pltpu.async_copy(rows_v.at[k], agg_out.at[c, ck[k]], ssem)
    pltpu.make_async_copy(rows_v.at[0], agg_out.at[c, ck[0]], ssem).wait()
    pltpu.async_copy(agg_sh.at[ck[4]], rows_v.at[0], gsem)
    pltpu.make_async_copy(agg_sh.at[ck[4]], rows_v.at[0], gsem).wait()
    pltpu.async_copy(rows_v.at[0], agg_out.at[c, ck[4]], ssem)
    for k in range(1, 5):
        pltpu.make_async_copy(rows_v.at[k % 4], agg_out.at[c, ck[k]], ssem).wait()

    @pl.when(c == 0)
    def _():
        for k in range(RPT // ZCH):
            base = s * RPT + k * ZCH
            pltpu.sync_copy(cnt_sh.at[pl.ds(base, ZCH)], zcnt_v)
            pltpu.sync_copy(zcnt_v, cnt_out.at[pl.ds(base, ZCH)])


def _sc_agg1(x2, e4, zrow, zcnt, ones):
    return pl.kernel(
        _sc_agg1_body,
        out_type=[
            jax.ShapeDtypeStruct((NC, NP, FH), jnp.float32),
            jax.ShapeDtypeStruct((NP, CW), jnp.float32),
        ],
        mesh=_sc_mesh(),
        compiler_params=pltpu.CompilerParams(use_tc_tiling_on_sc=False),
        scratch_types=[
            pltpu.VMEM((NB, B), jnp.int32),
            pltpu.VMEM((NB, B), jnp.int32),
            pltpu.VMEM((4, B, FH), jnp.float32),
            pltpu.VMEM((B, CW), jnp.float32),
            pltpu.VMEM((ZCH, FH), jnp.float32),
            pltpu.VMEM((ZCH, CW), jnp.float32),
            pltpu.VMEM_SHARED((NP, FH), jnp.float32),
            pltpu.VMEM_SHARED((NP, CW), jnp.float32),
            pltpu.SemaphoreType.DMA,
            pltpu.SemaphoreType.DMA,
            pltpu.SemaphoreType.DMA,
        ],
    )(x2, e4, zrow, zcnt, ones)


def _sc_agg2_body(p_hbm, edges_hbm, zrow_hbm,
                  agg_out,
                  src_v, dst_v, rows_v, zrow_v,
                  agg_sh, gsem, ssem):
    c = lax.axis_index("c")
    s = lax.axis_index("s")

    pltpu.sync_copy(zrow_hbm, zrow_v)
    pltpu.sync_copy(edges_hbm.at[0, s], src_v)
    pltpu.sync_copy(edges_hbm.at[1, s], dst_v)
    pad_fill = jnp.full((16,), PAD_DST, jnp.int32)
    for t in range((EPTP - EPT) // 16):
        dst_v[NB - 1, pl.ds(EPT - (NB - 1) * B + t * 16, 16)] = pad_fill

    for k in range(RPT // ZCH):
        base = s * RPT + k * ZCH
        pltpu.async_copy(zrow_v, agg_sh.at[pl.ds(base, ZCH)], gsem)
    for k in range(RPT // ZCH):
        base = s * RPT + k * ZCH
        pltpu.make_async_copy(zrow_v, agg_sh.at[pl.ds(base, ZCH)], gsem).wait()
    plsc.subcore_barrier()

    for pr in range(4):
        pltpu.async_copy(p_hbm.at[c].at[src_v.at[pr]], rows_v.at[pr], gsem)

    def body(i, carry):
        cur = lax.rem(i, 8)
        nxt = lax.rem(i + 4, 8)
        pltpu.make_async_copy(p_hbm.at[c].at[src_v.at[i]], rows_v.at[cur], gsem).wait()

        @pl.when(i > 3)
        def _():
            pltpu.make_async_copy(rows_v.at[nxt], agg_sh.at[dst_v.at[i]],
                                  ssem).wait()

        @pl.when(i + 4 < NB)
        def _():
            pltpu.async_copy(p_hbm.at[c].at[src_v.at[i + 4]], rows_v.at[nxt], gsem)

        pltpu.async_copy(rows_v.at[cur], agg_sh.at[dst_v.at[i]], ssem, add=True)
        return carry

    lax.fori_loop(0, NB, body, 0)
    for tail in range(NB - 4, NB):
        pltpu.make_async_copy(rows_v.at[tail % 8], agg_sh.at[dst_v.at[tail]],
                              ssem).wait()
    plsc.subcore_barrier()

    ck = [pl.ds(s * RPT + k * ZCH, ZCH) for k in range(RPT // ZCH)]
    for k in range(5):
        pltpu.async_copy(agg_sh.at[ck[k]], rows_v.at[k], gsem)
    for k in range(5):
        pltpu.make_async_copy(agg_sh.at[ck[k]], rows_v.at[k], gsem).wait()
        pltpu.async_copy(rows_v.at[k], agg_out.at[c, ck[k]], ssem)
    for k in range(5):
        pltpu.make_async_copy(rows_v.at[k], agg_out.at[c, ck[k]], ssem).wait()


def _sc_agg2(p2, e4, zrow):
    return pl.kernel(
        _sc_agg2_body,
        out_type=jax.ShapeDtypeStruct((NC, NP, CHP), jnp.float32),
        mesh=_sc_mesh(),
        compiler_params=pltpu.CompilerParams(use_tc_tiling_on_sc=False),
        scratch_types=[
            pltpu.VMEM((NB, B), jnp.int32),
            pltpu.VMEM((NB, B), jnp.int32),
            pltpu.VMEM((8, B, CHP), jnp.float32),
            pltpu.VMEM((ZCH, CHP), jnp.float32),
            pltpu.VMEM_SHARED((NP, CHP), jnp.float32),
            pltpu.SemaphoreType.DMA,
            pltpu.SemaphoreType.DMA,
        ],
    )(p2, e4, zrow)


RB = 1000


def _dot_t(a, w):
    return lax.dot_general(a, w, (((1,), (1,)), ((), ())),
                           preferred_element_type=jnp.float32)


def _tc_layer1_body(aL, aR, cn, x, w1, b1l, w2, b2l, p2_out, q_out):
    cnt = jnp.maximum(cn[...], 1.0)
    mean = jnp.concatenate([aL[0], aR[0]], axis=1) / cnt
    mx = jnp.concatenate([mean, x[...]], axis=1)
    h = jnp.maximum(_dot_t(mx, w1[...]) + b1l[...], 0.0)
    pq = _dot_t(h, w2[...])
    zpad = jnp.zeros((pq.shape[0], CHP - CH), jnp.float32)
    p2_out[0] = jnp.concatenate([pq[:, :CH], zpad], axis=1)
    p2_out[1] = jnp.concatenate([pq[:, CH:CLS], zpad], axis=1)
    q_out[...] = pq[:, CLS:] + b2l[...]


def _tc_layer1(agg_pp, cn, x, W1l, b1l, W1r, W2l, b2l, W2r):
    W1 = jnp.concatenate([W1l, W1r], axis=1)
    W2 = jnp.concatenate([W2l, W2r], axis=0)
    blk = lambda r, c: pl.BlockSpec((r, c), lambda i: (i, 0))
    full = lambda r, c: pl.BlockSpec((r, c), lambda i: (0, 0))
    return pl.pallas_call(
        _tc_layer1_body,
        grid=(N // RB,),
        in_specs=[
            pl.BlockSpec((1, RB, FH), lambda i: (0, i, 0)),
            pl.BlockSpec((1, RB, FH), lambda i: (1, i, 0)),
            blk(RB, 1), blk(RB, F_IN),
            full(HID, 2 * F_IN), full(1, HID),
            full(2 * CLS, HID), full(1, CLS),
        ],
        out_specs=[
            pl.BlockSpec((NC, RB, CHP), lambda i: (0, i, 0)),
            blk(RB, CLS),
        ],
        out_shape=[
            jax.ShapeDtypeStruct((NC, N, CHP), jnp.float32),
            jax.ShapeDtypeStruct((N, CLS), jnp.float32),
        ],
    )(agg_pp, agg_pp, cn, x, W1, b1l, W2, b2l)


def _tc_layer2_body(g0, g1, cn, q, out):
    agg = jnp.concatenate([g0[0, :, :CH], g1[0, :, :CH]], axis=1)
    cnt = jnp.maximum(cn[...], 1.0)
    z = agg / cnt + q[...]
    m = jnp.max(z, axis=1, keepdims=True)
    zs = z - m
    out[...] = zs - jnp.log(jnp.sum(jnp.exp(zs), axis=1, keepdims=True))


def _tc_layer2(agg2_pp, cn, q):
    blk = lambda r, c: pl.BlockSpec((r, c), lambda i: (i, 0))
    return pl.pallas_call(
        _tc_layer2_body,
        grid=(N // RB,),
        in_specs=[
            pl.BlockSpec((1, RB, CHP), lambda i: (0, i, 0)),
            pl.BlockSpec((1, RB, CHP), lambda i: (1, i, 0)),
            blk(RB, 1), blk(RB, CLS),
        ],
        out_specs=blk(RB, CLS),
        out_shape=jax.ShapeDtypeStruct((N, CLS), jnp.float32),
    )(agg2_pp, agg2_pp, cn, q)


def kernel(x, edge_index, W1l, b1l, W1r, W2l, b2l, W2r):
    e4 = jnp.pad(edge_index.reshape(2, NS, EPT),
                 ((0, 0), (0, 0), (0, EPTP - EPT))).reshape(2, NS, NB, B)
    x2 = jnp.stack([x[:, :FH], x[:, FH:]])

    zrow = jnp.zeros((ZCH, FH), jnp.float32)
    zcnt = jnp.zeros((ZCH, CW), jnp.float32)
    zrow2 = jnp.zeros((ZCH, CHP), jnp.float32)
    ones = jnp.ones((B, CW), jnp.float32)

    agg_pp, cnt_pp = _sc_agg1(x2, e4, zrow, zcnt, ones)
    cn = cnt_pp[:N, :1]
    p2, q = _tc_layer1(agg_pp, cn, x,
                       W1l, b1l.reshape(1, HID), W1r,
                       W2l, b2l.reshape(1, CLS), W2r)

    agg2_pp = _sc_agg2(p2, e4, zrow2)
    return _tc_layer2(agg2_pp, cn, q)
